# baseline (speedup 1.0000x reference)
import jax
import jax.numpy as jnp
from jax import lax
from jax.experimental import pallas as pl
from jax.experimental.pallas import tpu as pltpu
from jax.experimental.pallas import tpu_sc as plsc

D = 128
B = 16384
NC = 2
NS = 16
L = 16
NW = NC * NS
BPW = B // NW
CB = 256
NCHUNK = BPW // CB
GI = 128
NGATH = 2 * CB // GI


def _body(idxp_hbm, w_hbm, out_hbm,
          ia0, ia1, ia2, ia3, ib0, ib1, ib2, ib3,
          rows_a, rows_b, out_v, sem_a, sem_b):
    wid = lax.axis_index("s") * NC + lax.axis_index("c")
    base2 = wid * BPW * 2
    bufs = (((ia0, ia1, ia2, ia3), rows_a, sem_a),
            ((ib0, ib1, ib2, ib3), rows_b, sem_b))

    def issue(c, slot):
        idxs, rows, sem = bufs[slot]
        for j in range(NGATH):
            pltpu.sync_copy(
                idxp_hbm.at[pl.ds(base2 + c * 2 * CB + j * GI, GI)], idxs[j])
        for j in range(NGATH):
            pltpu.async_copy(w_hbm.at[idxs[j]],
                             rows.at[pl.ds(j * GI, GI)], sem)

    def wait(slot):
        idxs, rows, sem = bufs[slot]
        for j in range(NGATH):
            pltpu.make_async_copy(w_hbm.at[idxs[j]],
                                  rows.at[pl.ds(j * GI, GI)], sem).wait()

    lanes = lax.iota(jnp.int32, L)
    last_lane = lanes == (L - 1)

    issue(0, 0)
    for c in range(NCHUNK):
        slot = c % 2
        if c + 1 < NCHUNK:
            issue(c + 1, 1 - slot)
        wait(slot)
        _, rows, _ = bufs[slot]

        @plsc.parallel_loop(0, CB, 1, unroll=4)
        def _(e, rows=rows, c=c):
            acc0 = jnp.zeros((L,), jnp.float32)
            acc1 = jnp.zeros((L,), jnp.float32)
            for s in range(D // (2 * L)):
                x0 = rows[2 * e, pl.ds(s * 2 * L, 2 * L)]
                x1 = rows[2 * e + 1, pl.ds(s * 2 * L, 2 * L)]
                a0, b0 = plsc.unpack(x0, format=plsc.PackFormat.INTERLEAVED)
                a1, b1 = plsc.unpack(x1, format=plsc.PackFormat.INTERLEAVED)
                acc0 = acc0 + a0 * a1
                acc1 = acc1 + b0 * b1
            scn = plsc.cumsum(acc0 + acc1)
            pos = jnp.full((L,), c * CB + e, jnp.int32)
            plsc.store_scatter(out_v, [pos], scn, mask=last_lane)

    pltpu.sync_copy(out_v, out_hbm.at[pl.ds(wid * BPW, BPW)])


def kernel(X_idxs, W):
    idxp = X_idxs.reshape(-1).astype(jnp.int32)
    w_bf = W.astype(jnp.bfloat16)
    mesh = plsc.VectorSubcoreMesh(core_axis_name="c", subcore_axis_name="s")
    f = pl.kernel(
        _body,
        out_type=jax.ShapeDtypeStruct((B,), jnp.float32),
        mesh=mesh,
        compiler_params=pltpu.CompilerParams(
            needs_layout_passes=False, use_tc_tiling_on_sc=False),
        scratch_types=[
            pltpu.VMEM((GI,), jnp.int32),
            pltpu.VMEM((GI,), jnp.int32),
            pltpu.VMEM((GI,), jnp.int32),
            pltpu.VMEM((GI,), jnp.int32),
            pltpu.VMEM((GI,), jnp.int32),
            pltpu.VMEM((GI,), jnp.int32),
            pltpu.VMEM((GI,), jnp.int32),
            pltpu.VMEM((GI,), jnp.int32),
            pltpu.VMEM((2 * CB, D), jnp.bfloat16),
            pltpu.VMEM((2 * CB, D), jnp.bfloat16),
            pltpu.VMEM((BPW,), jnp.float32),
            pltpu.SemaphoreType.DMA,
            pltpu.SemaphoreType.DMA,
        ],
    )
    return f(idxp, w_bf)


# speedup vs baseline: 1.0102x; 1.0102x over previous
import jax
import jax.numpy as jnp
from jax import lax
from jax.experimental import pallas as pl
from jax.experimental.pallas import tpu as pltpu
from jax.experimental.pallas import tpu_sc as plsc

D = 128
B = 16384
NC = 2
NS = 16
L = 16
NW = NC * NS
BPW = B // NW
CB = 128
NCHUNK = BPW // CB
GI = 128
NGATH = 2 * CB // GI


def _body(idxp_hbm, w_hbm, out_hbm,
          ia0, ia1, ib0, ib1,
          rows_a, rows_b, out_v, sem_a, sem_b):
    wid = lax.axis_index("s") * NC + lax.axis_index("c")
    base2 = wid * BPW * 2
    bufs = (((ia0, ia1), rows_a, sem_a),
            ((ib0, ib1), rows_b, sem_b))

    def issue(c, slot):
        idxs, rows, sem = bufs[slot]
        for j in range(NGATH):
            pltpu.sync_copy(
                idxp_hbm.at[pl.ds(base2 + c * 2 * CB + j * GI, GI)], idxs[j])
        for j in range(NGATH):
            pltpu.async_copy(w_hbm.at[idxs[j]],
                             rows.at[pl.ds(j * GI, GI)], sem)

    def wait(slot):
        idxs, rows, sem = bufs[slot]
        for j in range(NGATH):
            pltpu.make_async_copy(w_hbm.at[idxs[j]],
                                  rows.at[pl.ds(j * GI, GI)], sem).wait()

    lanes = lax.iota(jnp.int32, L)
    last_lane = lanes == (L - 1)

    issue(0, 0)
    for c in range(NCHUNK):
        slot = c % 2
        if c + 1 < NCHUNK:
            issue(c + 1, 1 - slot)
        wait(slot)
        _, rows, _ = bufs[slot]

        @plsc.parallel_loop(0, CB, 1, unroll=4)
        def _(e, rows=rows, c=c):
            acc0 = jnp.zeros((L,), jnp.float32)
            acc1 = jnp.zeros((L,), jnp.float32)
            for s in range(D // (2 * L)):
                x0 = rows[2 * e, pl.ds(s * 2 * L, 2 * L)]
                x1 = rows[2 * e + 1, pl.ds(s * 2 * L, 2 * L)]
                a0, b0 = plsc.unpack(x0, format=plsc.PackFormat.INTERLEAVED)
                a1, b1 = plsc.unpack(x1, format=plsc.PackFormat.INTERLEAVED)
                acc0 = acc0 + a0 * a1
                acc1 = acc1 + b0 * b1
            scn = plsc.cumsum(acc0 + acc1)
            pos = jnp.full((L,), c * CB + e, jnp.int32)
            plsc.store_scatter(out_v, [pos], scn, mask=last_lane)

    pltpu.sync_copy(out_v, out_hbm.at[pl.ds(wid * BPW, BPW)])


def kernel(X_idxs, W):
    idxp = X_idxs.reshape(-1).astype(jnp.int32)
    w_bf = W.astype(jnp.bfloat16)
    mesh = plsc.VectorSubcoreMesh(core_axis_name="c", subcore_axis_name="s")
    f = pl.kernel(
        _body,
        out_type=jax.ShapeDtypeStruct((B,), jnp.float32),
        mesh=mesh,
        compiler_params=pltpu.CompilerParams(
            needs_layout_passes=False, use_tc_tiling_on_sc=False),
        scratch_types=[
            pltpu.VMEM((GI,), jnp.int32),
            pltpu.VMEM((GI,), jnp.int32),
            pltpu.VMEM((GI,), jnp.int32),
            pltpu.VMEM((GI,), jnp.int32),
            pltpu.VMEM((2 * CB, D), jnp.bfloat16),
            pltpu.VMEM((2 * CB, D), jnp.bfloat16),
            pltpu.VMEM((BPW,), jnp.float32),
            pltpu.SemaphoreType.DMA,
            pltpu.SemaphoreType.DMA,
        ],
    )
    return f(idxp, w_bf)


# speedup vs baseline: 1.3769x; 1.3630x over previous
import jax
import jax.numpy as jnp
from jax import lax
from jax.experimental import pallas as pl
from jax.experimental.pallas import tpu as pltpu
from jax.experimental.pallas import tpu_sc as plsc

D = 128
B = 16384
NC = 2
NS = 16
L = 16
NW = NC * NS
BPW = B // NW
CB = 128
NCHUNK = BPW // CB


def _body(idx0_hbm, idx1_hbm, w_hbm, out_hbm,
          i0a, i1a, i0b, i1b, r0a, r1a, r0b, r1b, out_v,
          s0a, s1a, s0b, s1b):
    wid = lax.axis_index("s") * NC + lax.axis_index("c")
    base = wid * BPW
    bufs = ((i0a, i1a, r0a, r1a, s0a, s1a),
            (i0b, i1b, r0b, r1b, s0b, s1b))

    def issue(c, slot):
        i0, i1, r0, r1, s0, s1 = bufs[slot]
        cbase = base + c * CB
        pltpu.sync_copy(idx0_hbm.at[pl.ds(cbase, CB)], i0)
        pltpu.sync_copy(idx1_hbm.at[pl.ds(cbase, CB)], i1)
        pltpu.async_copy(w_hbm.at[i0], r0, s0)
        pltpu.async_copy(w_hbm.at[i1], r1, s1)

    def wait(slot):
        i0, i1, r0, r1, s0, s1 = bufs[slot]
        pltpu.make_async_copy(w_hbm.at[i0], r0, s0).wait()
        pltpu.make_async_copy(w_hbm.at[i1], r1, s1).wait()

    lanes = lax.iota(jnp.int32, L)
    last_lane = lanes == (L - 1)

    issue(0, 0)
    for c in range(NCHUNK):
        slot = c % 2
        if c + 1 < NCHUNK:
            issue(c + 1, 1 - slot)
        wait(slot)
        _, _, r0, r1, _, _ = bufs[slot]

        @plsc.parallel_loop(0, CB, 1, unroll=4)
        def _(e, r0=r0, r1=r1, c=c):
            acc0 = jnp.zeros((L,), jnp.float32)
            acc1 = jnp.zeros((L,), jnp.float32)
            for s in range(D // (2 * L)):
                x0 = r0[e, pl.ds(s * 2 * L, 2 * L)]
                x1 = r1[e, pl.ds(s * 2 * L, 2 * L)]
                p = x0 * x1
                a, b = plsc.unpack(p, format=plsc.PackFormat.INTERLEAVED)
                acc0 = acc0 + a
                acc1 = acc1 + b
            scn = plsc.cumsum(acc0 + acc1)
            pos = jnp.full((L,), c * CB + e, jnp.int32)
            plsc.store_scatter(out_v, [pos], scn, mask=last_lane)

    pltpu.sync_copy(out_v, out_hbm.at[pl.ds(base, BPW)])


def kernel(X_idxs, W):
    idx0 = X_idxs[:, 0].astype(jnp.int32)
    idx1 = X_idxs[:, 1].astype(jnp.int32)
    w_bf = W.astype(jnp.bfloat16)
    mesh = plsc.VectorSubcoreMesh(core_axis_name="c", subcore_axis_name="s")
    f = pl.kernel(
        _body,
        out_type=jax.ShapeDtypeStruct((B,), jnp.float32),
        mesh=mesh,
        compiler_params=pltpu.CompilerParams(
            needs_layout_passes=False, use_tc_tiling_on_sc=False,
            disable_bounds_checks=True),
        scratch_types=[
            pltpu.VMEM((CB,), jnp.int32),
            pltpu.VMEM((CB,), jnp.int32),
            pltpu.VMEM((CB,), jnp.int32),
            pltpu.VMEM((CB,), jnp.int32),
            pltpu.VMEM((CB, D), jnp.bfloat16),
            pltpu.VMEM((CB, D), jnp.bfloat16),
            pltpu.VMEM((CB, D), jnp.bfloat16),
            pltpu.VMEM((CB, D), jnp.bfloat16),
            pltpu.VMEM((BPW,), jnp.float32),
            pltpu.SemaphoreType.DMA,
            pltpu.SemaphoreType.DMA,
            pltpu.SemaphoreType.DMA,
            pltpu.SemaphoreType.DMA,
        ],
    )
    return f(idx0, idx1, w_bf)


# speedup vs baseline: 1.3809x; 1.0029x over previous
import jax
import jax.numpy as jnp
from jax import lax
from jax.experimental import pallas as pl
from jax.experimental.pallas import tpu as pltpu
from jax.experimental.pallas import tpu_sc as plsc

D = 128
B = 16384
NC = 2
NS = 16
L = 16
NW = NC * NS
BPW = B // NW
CB = 128
NCHUNK = BPW // CB


def _body(idx0_hbm, idx1_hbm, w_hbm, out_hbm,
          i0a, i1a, i0b, i1b, r0a, r1a, r0b, r1b, out_v,
          s0a, s1a, s0b, s1b):
    wid = lax.axis_index("s") * NC + lax.axis_index("c")
    base = wid * BPW
    bufs = ((i0a, i1a, r0a, r1a, s0a, s1a),
            (i0b, i1b, r0b, r1b, s0b, s1b))

    def issue(c, slot):
        i0, i1, r0, r1, s0, s1 = bufs[slot]
        cbase = base + c * CB
        pltpu.sync_copy(idx0_hbm.at[pl.ds(cbase, CB)], i0)
        pltpu.sync_copy(idx1_hbm.at[pl.ds(cbase, CB)], i1)
        pltpu.async_copy(w_hbm.at[i0], r0, s0)
        pltpu.async_copy(w_hbm.at[i1], r1, s1)

    def wait(slot):
        i0, i1, r0, r1, s0, s1 = bufs[slot]
        pltpu.make_async_copy(w_hbm.at[i0], r0, s0).wait()
        pltpu.make_async_copy(w_hbm.at[i1], r1, s1).wait()

    lanes = lax.iota(jnp.int32, L)
    last_lane = lanes == (L - 1)

    issue(0, 0)
    for c in range(NCHUNK):
        slot = c % 2
        if c + 1 < NCHUNK:
            issue(c + 1, 1 - slot)
        wait(slot)
        _, _, r0, r1, _, _ = bufs[slot]

        @plsc.parallel_loop(0, CB, 1, unroll=4)
        def _(e, r0=r0, r1=r1, c=c):
            acc0 = jnp.zeros((L,), jnp.float32)
            acc1 = jnp.zeros((L,), jnp.float32)
            for s in range(D // (2 * L)):
                x0 = r0[e, pl.ds(s * 2 * L, 2 * L)]
                x1 = r1[e, pl.ds(s * 2 * L, 2 * L)]
                p = x0 * x1
                a, b = plsc.unpack(p, format=plsc.PackFormat.INTERLEAVED)
                acc0 = acc0 + a
                acc1 = acc1 + b
            scn = plsc.cumsum(acc0 + acc1)
            pos = jnp.full((L,), c * CB + e, jnp.int32)
            plsc.store_scatter(out_v, [pos], scn, mask=last_lane)

    pltpu.sync_copy(out_v, out_hbm.at[pl.ds(base, BPW)])


def kernel(X_idxs, W):
    idx0 = X_idxs[:, 0].astype(jnp.int32)
    idx1 = X_idxs[:, 1].astype(jnp.int32)
    w_bf = W.astype(jnp.bfloat16)
    mesh = plsc.VectorSubcoreMesh(core_axis_name="c", subcore_axis_name="s")
    f = pl.kernel(
        _body,
        out_type=jax.ShapeDtypeStruct((B,), jnp.float32),
        mesh=mesh,
        compiler_params=pltpu.CompilerParams(
            needs_layout_passes=False, use_tc_tiling_on_sc=False,
            disable_bounds_checks=True, skip_device_barrier=True),
        scratch_types=[
            pltpu.VMEM((CB,), jnp.int32),
            pltpu.VMEM((CB,), jnp.int32),
            pltpu.VMEM((CB,), jnp.int32),
            pltpu.VMEM((CB,), jnp.int32),
            pltpu.VMEM((CB, D), jnp.bfloat16),
            pltpu.VMEM((CB, D), jnp.bfloat16),
            pltpu.VMEM((CB, D), jnp.bfloat16),
            pltpu.VMEM((CB, D), jnp.bfloat16),
            pltpu.VMEM((BPW,), jnp.float32),
            pltpu.SemaphoreType.DMA,
            pltpu.SemaphoreType.DMA,
            pltpu.SemaphoreType.DMA,
            pltpu.SemaphoreType.DMA,
        ],
    )
    return f(idx0, idx1, w_bf)


# speedup vs baseline: 1.3840x; 1.0022x over previous
import jax
import jax.numpy as jnp
from jax import lax
from jax.experimental import pallas as pl
from jax.experimental.pallas import tpu as pltpu
from jax.experimental.pallas import tpu_sc as plsc

D = 128
B = 16384
NC = 2
NS = 16
L = 16
NW = NC * NS
BPW = B // NW
CB = 128
NCHUNK = BPW // CB


def _body(idx0_hbm, idx1_hbm, w_hbm, out_hbm,
          i0a, i1a, i0b, i1b, r0a, r1a, r0b, r1b, out_v,
          s0a, s1a, s0b, s1b):
    wid = lax.axis_index("s") * NC + lax.axis_index("c")
    base = wid * BPW
    bufs = ((i0a, i1a, r0a, r1a, s0a, s1a),
            (i0b, i1b, r0b, r1b, s0b, s1b))

    def issue(c, slot):
        i0, i1, r0, r1, s0, s1 = bufs[slot]
        cbase = base + c * CB
        pltpu.sync_copy(idx0_hbm.at[pl.ds(cbase, CB)], i0)
        pltpu.sync_copy(idx1_hbm.at[pl.ds(cbase, CB)], i1)
        pltpu.async_copy(w_hbm.at[i0], r0, s0)
        pltpu.async_copy(w_hbm.at[i1], r1, s1)

    def wait(slot):
        i0, i1, r0, r1, s0, s1 = bufs[slot]
        pltpu.make_async_copy(w_hbm.at[i0], r0, s0).wait()
        pltpu.make_async_copy(w_hbm.at[i1], r1, s1).wait()

    lanes = lax.iota(jnp.int32, L)
    last_lane = lanes == (L - 1)

    issue(0, 0)
    for c in range(NCHUNK):
        slot = c % 2
        if c + 1 < NCHUNK:
            issue(c + 1, 1 - slot)
        wait(slot)
        _, _, r0, r1, _, _ = bufs[slot]

        @plsc.parallel_loop(0, CB, 1, unroll=2)
        def _(e, r0=r0, r1=r1, c=c):
            acc0 = jnp.zeros((L,), jnp.float32)
            acc1 = jnp.zeros((L,), jnp.float32)
            for s in range(D // (2 * L)):
                x0 = r0[e, pl.ds(s * 2 * L, 2 * L)]
                x1 = r1[e, pl.ds(s * 2 * L, 2 * L)]
                p = x0 * x1
                a, b = plsc.unpack(p, format=plsc.PackFormat.INTERLEAVED)
                acc0 = acc0 + a
                acc1 = acc1 + b
            scn = plsc.cumsum(acc0 + acc1)
            pos = jnp.full((L,), c * CB + e, jnp.int32)
            plsc.store_scatter(out_v, [pos], scn, mask=last_lane)

    pltpu.sync_copy(out_v, out_hbm.at[pl.ds(base, BPW)])


def kernel(X_idxs, W):
    idx0 = X_idxs[:, 0].astype(jnp.int32)
    idx1 = X_idxs[:, 1].astype(jnp.int32)
    w_bf = W.astype(jnp.bfloat16)
    mesh = plsc.VectorSubcoreMesh(core_axis_name="c", subcore_axis_name="s")
    f = pl.kernel(
        _body,
        out_type=jax.ShapeDtypeStruct((B,), jnp.float32),
        mesh=mesh,
        compiler_params=pltpu.CompilerParams(
            needs_layout_passes=False, use_tc_tiling_on_sc=False,
            disable_bounds_checks=True, skip_device_barrier=True),
        scratch_types=[
            pltpu.VMEM((CB,), jnp.int32),
            pltpu.VMEM((CB,), jnp.int32),
            pltpu.VMEM((CB,), jnp.int32),
            pltpu.VMEM((CB,), jnp.int32),
            pltpu.VMEM((CB, D), jnp.bfloat16),
            pltpu.VMEM((CB, D), jnp.bfloat16),
            pltpu.VMEM((CB, D), jnp.bfloat16),
            pltpu.VMEM((CB, D), jnp.bfloat16),
            pltpu.VMEM((BPW,), jnp.float32),
            pltpu.SemaphoreType.DMA,
            pltpu.SemaphoreType.DMA,
            pltpu.SemaphoreType.DMA,
            pltpu.SemaphoreType.DMA,
        ],
    )
    return f(idx0, idx1, w_bf)
